# baseline (speedup 1.0000x reference)
import functools
import math

import jax
import jax.numpy as jnp
from jax import lax
from jax.experimental import pallas as pl
from jax.experimental.pallas import tpu as pltpu
from jax.experimental.pallas import tpu_sc as plsc

N_HID = 62
MAX_LEN = 3000
N_ROWS = 4096
D_PAD = 128

_W_HMS = (3600.0 / 3661.0, 60.0 / 3661.0, 1.0 / 3661.0)

_NC = 2
_NS = 16
_NW = _NC * _NS
_RPW = N_ROWS // _NW
_HALF = _RPW // 2


def _tc_table_body(table_ref, w_ref, b_ref, t_ref, out_ref, tt_ref):
    prod = lax.dot_general(
        table_ref[...], w_ref[...],
        (((1,), (1,)), ((), ())),
        preferred_element_type=jnp.float32,
    )
    h = prod + b_ref[...]
    out_ref[...] = jnp.concatenate(
        [h, jnp.zeros((MAX_LEN, D_PAD - N_HID), jnp.float32)], axis=1)
    tt_ref[...] = t_ref[...].T


_tc_table = pl.pallas_call(
    _tc_table_body,
    out_shape=[
        jax.ShapeDtypeStruct((MAX_LEN, D_PAD), jnp.float32),
        jax.ShapeDtypeStruct((3, N_ROWS), jnp.int32),
    ],
)


def _sc_body(t_hbm, tw_hbm, out_hbm, tv, rows, acc, hsem0, hsem1, osem):
    wid = lax.axis_index("s") * _NC + lax.axis_index("c")
    base = wid * _RPW
    hsems = (hsem0, hsem1)

    with jax.named_scope("t_stage"):
        pltpu.sync_copy(t_hbm.at[:, pl.ds(base, _RPW)], tv)

    with jax.named_scope("gather_issue"):
        cps = []
        for h in range(2):
            for k in range(3):
                cps.append(pltpu.async_copy(
                    tw_hbm.at[tv.at[k, pl.ds(h * _HALF, _HALF)]],
                    rows.at[k, pl.ds(h * _HALF, _HALF)],
                    hsems[h]))

    def block_body(i2, carry):
        for u in range(4):
            i = i2 * 4 + u
            for off in (0, 16, 32, N_HID - 16):
                s = pl.ds(off, 16)
                acc[i, s] = (_W_HMS[0] * rows[0, i, s]
                             + _W_HMS[1] * rows[1, i, s]
                             + _W_HMS[2] * rows[2, i, s])
        return carry

    ocps = []
    for h in range(2):
        with jax.named_scope(f"wait_half{h}"):
            for k in range(3):
                cps[3 * h + k].wait()
        with jax.named_scope(f"compute{h}"):
            lax.fori_loop(h * _HALF // 4, (h + 1) * _HALF // 4, block_body, 0)
        with jax.named_scope(f"out_issue{h}"):
            ocps.append(pltpu.async_copy(
                acc.at[pl.ds(h * _HALF, _HALF)],
                out_hbm.at[pl.ds(base + h * _HALF, _HALF)],
                osem))
    with jax.named_scope("out_drain"):
        for cp in ocps:
            cp.wait()


@functools.cache
def _sc_gather():
    return pl.kernel(
        _sc_body,
        out_type=jax.ShapeDtypeStruct((N_ROWS, N_HID), jnp.float32),
        mesh=plsc.VectorSubcoreMesh(core_axis_name="c", subcore_axis_name="s"),
        scratch_types=[
            pltpu.VMEM((3, _RPW), jnp.int32),
            pltpu.VMEM((3, _RPW, D_PAD), jnp.float32),
            pltpu.VMEM((_RPW, N_HID), jnp.float32),
            pltpu.SemaphoreType.DMA,
            pltpu.SemaphoreType.DMA,
            pltpu.SemaphoreType.DMA,
        ],
    )


def kernel(t, table, W, b):
    tw, tt = _tc_table(table, W, b.reshape(1, N_HID), t)
    return _sc_gather()(tt, tw)


# speedup vs baseline: 1.1152x; 1.1152x over previous
import functools
import math

import jax
import jax.numpy as jnp
from jax import lax
from jax.experimental import pallas as pl
from jax.experimental.pallas import tpu as pltpu
from jax.experimental.pallas import tpu_sc as plsc

N_HID = 62
MAX_LEN = 3000
N_ROWS = 4096
D_PAD = 128

_W_HMS = (3600.0 / 3661.0, 60.0 / 3661.0, 1.0 / 3661.0)

_NC = 2
_NS = 16
_NW = _NC * _NS
_RPW = N_ROWS // _NW
_HALF = _RPW // 2


def _tc_table_body(table_ref, w_ref, b_ref, out_ref):
    prod = lax.dot_general(
        table_ref[...], w_ref[...],
        (((1,), (1,)), ((), ())),
        preferred_element_type=jnp.float32,
    )
    h = prod + b_ref[...]
    out_ref[...] = jnp.concatenate(
        [h, jnp.zeros((MAX_LEN, D_PAD - N_HID), jnp.float32)], axis=1)


_tc_table = pl.pallas_call(
    _tc_table_body,
    out_shape=jax.ShapeDtypeStruct((MAX_LEN, D_PAD), jnp.float32),
)


def _sc_body(t_hbm, tw_hbm, out_hbm, tv, rows, acc, hsem0, hsem1, osem):
    wid = lax.axis_index("s") * _NC + lax.axis_index("c")
    base = wid * _RPW
    hsems = (hsem0, hsem1)

    pltpu.sync_copy(t_hbm.at[:, pl.ds(base, _RPW)], tv)

    cps = []
    for h in range(2):
        for k in range(3):
            cps.append(pltpu.async_copy(
                tw_hbm.at[tv.at[k, pl.ds(h * _HALF, _HALF)]],
                rows.at[k, pl.ds(h * _HALF, _HALF)],
                hsems[h]))

    def block_body(i2, carry):
        for u in range(4):
            i = i2 * 4 + u
            for off in (0, 16, 32, N_HID - 16):
                s = pl.ds(off, 16)
                acc[i, s] = (_W_HMS[0] * rows[0, i, s]
                             + _W_HMS[1] * rows[1, i, s]
                             + _W_HMS[2] * rows[2, i, s])
        return carry

    ocps = []
    for h in range(2):
        for k in range(3):
            cps[3 * h + k].wait()
        lax.fori_loop(h * _HALF // 4, (h + 1) * _HALF // 4, block_body, 0)
        ocps.append(pltpu.async_copy(
            acc.at[pl.ds(h * _HALF, _HALF)],
            out_hbm.at[pl.ds(base + h * _HALF, _HALF)],
            osem))
    for cp in ocps:
        cp.wait()


@functools.cache
def _sc_gather():
    return pl.kernel(
        _sc_body,
        out_type=jax.ShapeDtypeStruct((N_ROWS, N_HID), jnp.float32),
        mesh=plsc.VectorSubcoreMesh(core_axis_name="c", subcore_axis_name="s"),
        scratch_types=[
            pltpu.VMEM((3, _RPW), jnp.int32),
            pltpu.VMEM((3, _RPW, D_PAD), jnp.float32),
            pltpu.VMEM((_RPW, N_HID), jnp.float32),
            pltpu.SemaphoreType.DMA,
            pltpu.SemaphoreType.DMA,
            pltpu.SemaphoreType.DMA,
        ],
    )


def kernel(t, table, W, b):
    tw = _tc_table(table, W, b.reshape(1, N_HID))
    return _sc_gather()(t.T, tw)
